# baseline (speedup 1.0000x reference)
import functools

import jax
import jax.numpy as jnp
from jax import lax
from jax.experimental import pallas as pl
from jax.experimental.pallas import tpu as pltpu
from jax.experimental.pallas import tpu_sc as plsc

N, E, NFEAT, NHID = 10000, 320000, 128, 128
BN_EPS = 1e-5
SN_EPS = 1e-12

NC, NS = 2, 16
NW = NC * NS
CHUNK = 128
NCHUNK = 80
EPT = NCHUNK * CHUNK
E_PAD = EPT * NW
NHALF = 2
CPH = NCHUNK // NHALF
PPH = CPH // 2
N_SC = 10240
RPT = N_SC // NS


def _sc_aggregate_body(x_hbm, src_hbm, dst_hbm, zeros_hbm, out_hbm,
                       srcall, dstall, rows_a, rows_b, agg, sem_a, sem_b):
    c = lax.axis_index("c")
    s = lax.axis_index("s")
    row0 = s * RPT

    @pl.when(c == 0)
    def _():
        pltpu.sync_copy(x_hbm.at[pl.ds(row0, RPT)],
                        agg.at[pl.ds(row0, RPT)])

    @pl.when(c != 0)
    def _():
        pltpu.sync_copy(zeros_hbm, agg.at[pl.ds(row0, RPT)])

    wid = s * NC + c
    plsc.subcore_barrier()

    for h in range(NHALF):
        pltpu.sync_copy(src_hbm.at[wid, pl.ds(h * CPH, CPH)], srcall)
        pltpu.sync_copy(dst_hbm.at[wid, pl.ds(h * CPH, CPH)], dstall)
        pltpu.async_copy(x_hbm.at[srcall.at[0]], rows_a, sem_a)

        def pair(k, carry):
            j0 = 2 * k
            j1 = 2 * k + 1
            pltpu.async_copy(x_hbm.at[srcall.at[j1]], rows_b, sem_b)
            pltpu.make_async_copy(x_hbm.at[srcall.at[j0]], rows_a,
                                  sem_a).wait()
            pltpu.sync_copy(rows_a, agg.at[dstall.at[j0]], add=True)

            @pl.when(k < PPH - 1)
            def _():
                pltpu.async_copy(x_hbm.at[srcall.at[j0 + 2]], rows_a, sem_a)

            pltpu.make_async_copy(x_hbm.at[srcall.at[j1]], rows_b,
                                  sem_b).wait()
            pltpu.sync_copy(rows_b, agg.at[dstall.at[j1]], add=True)
            return carry

        lax.fori_loop(0, PPH, pair, 0)
    plsc.subcore_barrier()

    pltpu.sync_copy(agg.at[pl.ds(row0, RPT)],
                    out_hbm.at[c, pl.ds(row0, RPT)])


@functools.cache
def _get_sc_aggregate():
    mesh = plsc.VectorSubcoreMesh(core_axis_name="c", subcore_axis_name="s",
                                  num_cores=NC, num_subcores=NS)
    return pl.kernel(
        _sc_aggregate_body,
        out_type=jax.ShapeDtypeStruct((NC, N_SC, NFEAT), jnp.float32),
        mesh=mesh,
        scratch_types=[
            pltpu.VMEM((CPH, CHUNK), jnp.int32),
            pltpu.VMEM((CPH, CHUNK), jnp.int32),
            pltpu.VMEM((CHUNK, NFEAT), jnp.float32),
            pltpu.VMEM((CHUNK, NFEAT), jnp.float32),
            pltpu.VMEM_SHARED((N_SC, NFEAT), jnp.float32),
            pltpu.SemaphoreType.DMA,
            pltpu.SemaphoreType.DMA,
        ],
    )


_PREC = lax.Precision.HIGHEST


def _spectral(W, u):
    vT = lax.dot_general(u, W, (((1,), (0,)), ((), ())),
                         precision=_PREC)
    v = vT / (jnp.sqrt(jnp.sum(vT * vT)) + SN_EPS)
    Wv = lax.dot_general(v, W, (((1,), (1,)), ((), ())),
                         precision=_PREC)
    u_new = Wv / (jnp.sqrt(jnp.sum(Wv * Wv)) + SN_EPS)
    sigma = jnp.sum(u_new * Wv)
    return W * (1.0 / sigma)


BLK = 2000
GRID = N // BLK


def _tc1_body(agg_ref, w1_ref, b1_ref, u1_ref, h_ref, s_ref, q_ref):
    i = pl.program_id(0)
    W1n = _spectral(w1_ref[...], u1_ref[...])
    h0 = agg_ref[0] + agg_ref[1]
    h = lax.dot_general(h0, W1n, (((1,), (1,)), ((), ())), precision=_PREC)
    h = jnp.maximum(h + b1_ref[...], 0.0)
    h_ref[...] = h

    @pl.when(i == 0)
    def _():
        s_ref[...] = jnp.zeros_like(s_ref)
        q_ref[...] = jnp.zeros_like(q_ref)

    s_ref[...] += jnp.sum(h, axis=0, keepdims=True)
    q_ref[...] += jnp.sum(h * h, axis=0, keepdims=True)


_tc1 = pl.pallas_call(
    _tc1_body,
    grid=(GRID,),
    in_specs=[
        pl.BlockSpec((NC, BLK, NFEAT), lambda i: (0, i, 0)),
        pl.BlockSpec((NHID, NFEAT), lambda i: (0, 0)),
        pl.BlockSpec((1, NHID), lambda i: (0, 0)),
        pl.BlockSpec((1, NHID), lambda i: (0, 0)),
    ],
    out_specs=[
        pl.BlockSpec((BLK, NHID), lambda i: (i, 0)),
        pl.BlockSpec((1, NHID), lambda i: (0, 0)),
        pl.BlockSpec((1, NHID), lambda i: (0, 0)),
    ],
    out_shape=[
        jax.ShapeDtypeStruct((N, NHID), jnp.float32),
        jax.ShapeDtypeStruct((1, NHID), jnp.float32),
        jax.ShapeDtypeStruct((1, NHID), jnp.float32),
    ],
)


def _tc2_body(h_ref, s_ref, q_ref, g_ref, be_ref, w2_ref, b2_ref, u2_ref,
              out_ref):
    W2n = _spectral(w2_ref[...], u2_ref[...])
    mean = s_ref[...] * (1.0 / N)
    var = q_ref[...] * (1.0 / N) - mean * mean
    sc = g_ref[...] * lax.rsqrt(var + BN_EPS)
    W2eff = W2n * sc
    cvec = lax.dot_general(be_ref[...] - mean * sc, W2n,
                           (((1,), (1,)), ((), ())), precision=_PREC)
    cvec = cvec + b2_ref[...]
    out = lax.dot_general(h_ref[...], W2eff, (((1,), (1,)), ((), ())),
                          precision=_PREC)
    out_ref[...] = out + cvec


_tc2 = pl.pallas_call(
    _tc2_body,
    grid=(GRID,),
    in_specs=[
        pl.BlockSpec((BLK, NHID), lambda i: (i, 0)),
        pl.BlockSpec((1, NHID), lambda i: (0, 0)),
        pl.BlockSpec((1, NHID), lambda i: (0, 0)),
        pl.BlockSpec((1, NHID), lambda i: (0, 0)),
        pl.BlockSpec((1, NHID), lambda i: (0, 0)),
        pl.BlockSpec((NHID, NHID), lambda i: (0, 0)),
        pl.BlockSpec((1, NHID), lambda i: (0, 0)),
        pl.BlockSpec((1, NHID), lambda i: (0, 0)),
    ],
    out_specs=pl.BlockSpec((BLK, NHID), lambda i: (i, 0)),
    out_shape=jax.ShapeDtypeStruct((N, NHID), jnp.float32),
)


def kernel(x, edge_index, W1, b1, gamma, beta, W2, b2, u1, u2):
    src = edge_index[0]
    dst = edge_index[1]
    pad = E_PAD - E
    src_p = jnp.concatenate([src, jnp.full((pad,), N, jnp.int32)])
    dst_p = jnp.concatenate([dst, jnp.zeros((pad,), jnp.int32)])
    src_p = src_p.reshape(NW, NCHUNK, CHUNK)
    dst_p = dst_p.reshape(NW, NCHUNK, CHUNK)
    x_p = jnp.concatenate([x, jnp.zeros((N_SC - N, NFEAT), x.dtype)], axis=0)
    zeros_seed = jnp.zeros((RPT, NFEAT), jnp.float32)

    aggs = _get_sc_aggregate()(x_p, src_p, dst_p, zeros_seed)

    relu_h, sums, sumsq = _tc1(aggs, W1, b1.reshape(1, -1), u1.reshape(1, -1))
    out = _tc2(relu_h, sums, sumsq, gamma.reshape(1, -1), beta.reshape(1, -1),
               W2, b2.reshape(1, -1), u2.reshape(1, -1))
    return out
